# baseline (speedup 1.0000x reference)
import jax
import jax.numpy as jnp
from jax.experimental import pallas as pl
from jax.experimental.pallas import tpu as pltpu


def _combine(d_ref, f_ref, o_ref):
    o_ref[...] = f_ref[...] + 0.0 * d_ref[0]


def kernel(history_data, future_data, batch_seen, epoch, train, dummy_param):
    return pl.pallas_call(
        _combine,
        out_shape=jax.ShapeDtypeStruct(future_data.shape, jnp.float32),
        in_specs=[
            pl.BlockSpec(memory_space=pltpu.SMEM),
            pl.BlockSpec(memory_space=pltpu.VMEM),
        ],
    )(dummy_param, future_data)
